# Initial kernel scaffold; baseline (speedup 1.0000x reference)
import jax
import jax.numpy as jnp
from jax.experimental import pallas as pl


def kernel(xyz1, xyz2):
    raise NotImplementedError("write your pallas kernel here")


# speedup vs baseline: 5.2976x; 5.2976x over previous
import functools

import jax
import jax.numpy as jnp
from jax import lax
from jax.experimental import pallas as pl
from jax.experimental.pallas import tpu as pltpu
from jax.experimental.pallas import tpu_sc as plsc

B = 4
N = 4096
M = 4096
K = 32
L = 16
ROWS_PER_W = (B * M) // 32
CHUNKS = M // ROWS_PER_W
BUF = 64
FLUSH_AT = BUF - 16

_INF = float("inf")


def _sort16(k, v):
    return plsc.sort_key_val(k, v)


def _merge16(ak, ai, bk, bi):
    rk = lax.rev(bk, (0,))
    ri = lax.rev(bi, (0,))
    m = ak <= rk
    lok = jnp.where(m, ak, rk)
    loi = jnp.where(m, ai, ri)
    hik = jnp.where(m, rk, ak)
    hii = jnp.where(m, ri, ai)
    o0k, o0i = _sort16(lok, loi)
    o1k, o1i = _sort16(hik, hii)
    return o0k, o0i, o1k, o1i


def _low32(a0k, a0i, a1k, a1i, b0k, b0i, b1k, b1i):
    rb0k = lax.rev(b1k, (0,))
    rb0i = lax.rev(b1i, (0,))
    rb1k = lax.rev(b0k, (0,))
    rb1i = lax.rev(b0i, (0,))
    m0 = a0k <= rb0k
    c0k = jnp.where(m0, a0k, rb0k)
    c0i = jnp.where(m0, a0i, rb0i)
    m1 = a1k <= rb1k
    c1k = jnp.where(m1, a1k, rb1k)
    c1i = jnp.where(m1, a1i, rb1i)
    m = c0k <= c1k
    lk = jnp.where(m, c0k, c1k)
    li = jnp.where(m, c0i, c1i)
    hk = jnp.where(m, c1k, c0k)
    hi = jnp.where(m, c1i, c0i)
    o0k, o0i = _sort16(lk, li)
    o1k, o1i = _sort16(hk, hi)
    return o0k, o0i, o1k, o1i


def _knn_body(x1_hbm, x2_hbm, out_hbm, cx, cy, cz, qx, qy, qz,
              bufk, bufi, rk_ref, ri_ref, cnt_ref, t_ref, outv):
    info = plsc.get_sparse_core_info()
    nc = info.num_cores
    wid = lax.axis_index("s") * nc + lax.axis_index("c")
    b = wid // CHUNKS
    chunk = wid % CHUNKS
    row0 = chunk * ROWS_PER_W

    pltpu.sync_copy(x1_hbm.at[b * 3 + 0], cx)
    pltpu.sync_copy(x1_hbm.at[b * 3 + 1], cy)
    pltpu.sync_copy(x1_hbm.at[b * 3 + 2], cz)
    pltpu.sync_copy(x2_hbm.at[b * 3 + 0, pl.ds(row0, ROWS_PER_W)], qx)
    pltpu.sync_copy(x2_hbm.at[b * 3 + 1, pl.ds(row0, ROWS_PER_W)], qy)
    pltpu.sync_copy(x2_hbm.at[b * 3 + 2, pl.ds(row0, ROWS_PER_W)], qz)

    iota = lax.iota(jnp.int32, L)
    inf_vec = jnp.full((L,), _INF, jnp.float32)
    zero_vec = jnp.zeros((L,), jnp.int32)

    def flush():
        b0k = bufk[pl.ds(0, L)]
        b1k = bufk[pl.ds(L, L)]
        b2k = bufk[pl.ds(2 * L, L)]
        b3k = bufk[pl.ds(3 * L, L)]
        b0i = bufi[pl.ds(0, L)]
        b1i = bufi[pl.ds(L, L)]
        b2i = bufi[pl.ds(2 * L, L)]
        b3i = bufi[pl.ds(3 * L, L)]
        s0k, s0i = _sort16(b0k, b0i)
        s1k, s1i = _sort16(b1k, b1i)
        s2k, s2i = _sort16(b2k, b2i)
        s3k, s3i = _sort16(b3k, b3i)
        a = _merge16(s0k, s0i, s1k, s1i)
        c = _merge16(s2k, s2i, s3k, s3i)
        d = _low32(*a, *c)
        r = (rk_ref[pl.ds(0, L)], ri_ref[pl.ds(0, L)],
             rk_ref[pl.ds(L, L)], ri_ref[pl.ds(L, L)])
        n0k, n0i, n1k, n1i = _low32(*d, *r)
        rk_ref[pl.ds(0, L)] = n0k
        ri_ref[pl.ds(0, L)] = n0i
        rk_ref[pl.ds(L, L)] = n1k
        ri_ref[pl.ds(L, L)] = n1i
        t_ref[...] = jnp.full((L,), jnp.max(n1k), jnp.float32)
        bufk[pl.ds(0, L)] = inf_vec
        bufk[pl.ds(L, L)] = inf_vec
        bufk[pl.ds(2 * L, L)] = inf_vec
        bufk[pl.ds(3 * L, L)] = inf_vec
        cnt_ref[...] = zero_vec

    def row_body(m, _):
        rk_ref[pl.ds(0, L)] = inf_vec
        rk_ref[pl.ds(L, L)] = inf_vec
        ri_ref[pl.ds(0, L)] = zero_vec
        ri_ref[pl.ds(L, L)] = zero_vec
        bufk[pl.ds(0, L)] = inf_vec
        bufk[pl.ds(L, L)] = inf_vec
        bufk[pl.ds(2 * L, L)] = inf_vec
        bufk[pl.ds(3 * L, L)] = inf_vec
        cnt_ref[...] = zero_vec
        t_ref[...] = inf_vec

        idxm = jnp.full((L,), m, jnp.int32)
        qxv = plsc.load_gather(qx, [idxm])
        qyv = plsc.load_gather(qy, [idxm])
        qzv = plsc.load_gather(qz, [idxm])

        def cand_body(j, _):
            base = j * L
            dx = cx[pl.ds(base, L)] - qxv
            dy = cy[pl.ds(base, L)] - qyv
            dz = cz[pl.ds(base, L)] - qzv
            d = dx * dx + dy * dy + dz * dz
            keep = d <= t_ref[...]
            cnt = cnt_ref[...]
            inc = keep.astype(jnp.int32)
            pos = plsc.cumsum(inc)
            tgt = pos + (cnt - 1)
            plsc.store_scatter(bufk, [tgt], d, mask=keep)
            plsc.store_scatter(bufi, [tgt], iota + base, mask=keep)
            ncnt = cnt + plsc.all_reduce_population_count(keep)
            cnt_ref[...] = ncnt
            pl.when(jnp.any(ncnt >= FLUSH_AT))(flush)
            return 0

        lax.fori_loop(0, N // L, cand_body, 0)
        flush()
        outv[pl.ds(m * K, L)] = ri_ref[pl.ds(0, L)]
        outv[pl.ds(m * K + L, L)] = ri_ref[pl.ds(L, L)]
        return 0

    lax.fori_loop(0, ROWS_PER_W, row_body, 0)
    pltpu.sync_copy(outv, out_hbm.at[pl.ds((b * M + row0) * K,
                                           ROWS_PER_W * K)])


@jax.jit
def _knn_sc(x1t, x2t):
    mesh = plsc.VectorSubcoreMesh(core_axis_name="c", subcore_axis_name="s")
    f = functools.partial(
        pl.kernel,
        out_type=jax.ShapeDtypeStruct((B * M * K,), jnp.int32),
        mesh=mesh,
        compiler_params=pltpu.CompilerParams(needs_layout_passes=False),
        scratch_types=[
            pltpu.VMEM((N,), jnp.float32),
            pltpu.VMEM((N,), jnp.float32),
            pltpu.VMEM((N,), jnp.float32),
            pltpu.VMEM((ROWS_PER_W,), jnp.float32),
            pltpu.VMEM((ROWS_PER_W,), jnp.float32),
            pltpu.VMEM((ROWS_PER_W,), jnp.float32),
            pltpu.VMEM((BUF,), jnp.float32),
            pltpu.VMEM((BUF,), jnp.int32),
            pltpu.VMEM((K,), jnp.float32),
            pltpu.VMEM((K,), jnp.int32),
            pltpu.VMEM((L,), jnp.int32),
            pltpu.VMEM((L,), jnp.float32),
            pltpu.VMEM((ROWS_PER_W * K,), jnp.int32),
        ],
    )(_knn_body)
    return f(x1t, x2t)


def kernel(xyz1, xyz2):
    x1t = xyz1.transpose(0, 2, 1).reshape(B * 3, N)
    x2t = xyz2.transpose(0, 2, 1).reshape(B * 3, M)
    out = _knn_sc(x1t, x2t)
    return out.reshape(B, M, K, 1)
